# baseline (speedup 1.0000x reference)
import functools

import jax
import jax.numpy as jnp
from jax.experimental import pallas as pl
from jax.experimental.pallas import tpu as pltpu

_CHUNK = 512
_NBUF = 4


def _router_kernel(x_hbm, w_ref, o_ref, buf, sems):
    n_chunks = x_hbm.shape[0] // _CHUNK
    w = w_ref[...]

    def _copy(i, slot):
        return pltpu.make_async_copy(
            x_hbm.at[pl.ds(i * _CHUNK, _CHUNK), :],
            buf.at[slot],
            sems.at[slot],
        )

    for s in range(_NBUF):
        _copy(s, s).start()

    def step(i, carry):
        slot = jax.lax.rem(i, _NBUF)
        _copy(i, slot).wait()
        logits = jax.lax.dot_general(
            buf[slot], w,
            dimension_numbers=(((1,), (1,)), ((), ())),
            preferred_element_type=jnp.float32,
        )
        m = jnp.max(logits, axis=-1, keepdims=True)
        e = jnp.exp(logits - m)
        o_ref[pl.ds(i * _CHUNK, _CHUNK), :] = e / jnp.sum(e, axis=-1, keepdims=True)

        @pl.when(i + _NBUF < n_chunks)
        def _():
            _copy(i + _NBUF, slot).start()

        return carry

    jax.lax.fori_loop(0, n_chunks, step, 0)


@functools.partial(jax.jit, static_argnames=("interpret",))
def kernel(x, expert_embeddings, interpret=False):
    B, S, H = x.shape
    E = expert_embeddings.shape[0]
    rows = B * S
    x2 = x.reshape(rows, H)
    out = pl.pallas_call(
        _router_kernel,
        in_specs=[
            pl.BlockSpec(memory_space=pltpu.MemorySpace.HBM),
            pl.BlockSpec((E, H), lambda: (0, 0)),
        ],
        out_specs=pl.BlockSpec((rows, E), lambda: (0, 0)),
        out_shape=jax.ShapeDtypeStruct((rows, E), jnp.float32),
        scratch_shapes=[
            pltpu.VMEM((_NBUF, _CHUNK, H), jnp.float32),
            pltpu.SemaphoreType.DMA((_NBUF,)),
        ],
        interpret=interpret,
    )(x2, expert_embeddings)
    return out.reshape(B, S, E)


# speedup vs baseline: 1.0411x; 1.0411x over previous
import functools

import jax
import jax.numpy as jnp
from jax.experimental import pallas as pl
from jax.experimental.pallas import tpu as pltpu

_TILE = 512


def _router_kernel(xa_ref, xb_ref, w_ref, o_ref):
    w = w_ref[...]

    def _softmax_dot(x_blk):
        logits = jax.lax.dot_general(
            x_blk, w,
            dimension_numbers=(((1,), (1,)), ((), ())),
            preferred_element_type=jnp.float32,
        )
        m = jnp.max(logits, axis=-1, keepdims=True)
        e = jnp.exp(logits - m)
        return e / jnp.sum(e, axis=-1, keepdims=True)

    o_ref[0] = _softmax_dot(xa_ref[...])
    o_ref[1] = _softmax_dot(xb_ref[...])


@functools.partial(jax.jit, static_argnames=("interpret",))
def kernel(x, expert_embeddings, interpret=False):
    B, S, H = x.shape
    E = expert_embeddings.shape[0]
    rows = B * S
    half_steps = rows // (2 * _TILE)
    x2 = x.reshape(rows, H)
    out = pl.pallas_call(
        _router_kernel,
        grid=(half_steps,),
        in_specs=[
            pl.BlockSpec((_TILE, H), lambda i: (i, 0)),
            pl.BlockSpec((_TILE, H), lambda i: (i + half_steps, 0)),
            pl.BlockSpec((E, H), lambda i: (0, 0)),
        ],
        out_specs=pl.BlockSpec((2, _TILE, E), lambda i: (0, i, 0)),
        out_shape=jax.ShapeDtypeStruct((2, rows // 2, E), jnp.float32),
        compiler_params=pltpu.CompilerParams(
            dimension_semantics=("arbitrary",),
        ),
        interpret=interpret,
    )(x2, x2, expert_embeddings)
    return out.reshape(B, S, E)
